# baseline (speedup 1.0000x reference)
import jax
import jax.numpy as jnp
from jax.experimental import pallas as pl
from jax.experimental.pallas import tpu as pltpu

_ROWS = 512
_DEPTH = 12


def _qtab_kernel(in_hbm, mask_ref, table_ref, out_ref, bufs, sems):
    nchunk = in_hbm.shape[0] // _ROWS
    table = table_ref[...].astype(jnp.bfloat16)

    def start(chunk, slot):
        pltpu.make_async_copy(
            in_hbm.at[pl.ds(chunk * _ROWS, _ROWS), :],
            bufs.at[slot],
            sems.at[slot],
        ).start(priority=slot % 2)

    for slot in range(min(_DEPTH, nchunk)):
        start(slot, slot)
    for i in range(nchunk):
        slot = i % _DEPTH
        pltpu.make_async_copy(
            in_hbm.at[pl.ds(i * _ROWS, _ROWS), :],
            bufs.at[slot],
            sems.at[slot],
        ).wait()
        a = bufs[slot].astype(jnp.bfloat16)
        acc = jnp.dot(a, table, preferred_element_type=jnp.float32)
        ztab = table_ref[...].astype(jnp.bfloat16) * jnp.bfloat16(0.0)
        for _ in range(3):
            acc = acc + jnp.dot(a, ztab, preferred_element_type=jnp.float32)
        out_ref[pl.ds(i * _ROWS, _ROWS), :] = (
            acc + mask_ref[pl.ds(i * _ROWS, _ROWS), :]
        )
        nxt = i + _DEPTH
        if nxt < nchunk:
            start(nxt, slot)


def kernel(inputs, mask, table):
    B, K = inputs.shape
    N = table.shape[1]
    return pl.pallas_call(
        _qtab_kernel,
        in_specs=[
            pl.BlockSpec(memory_space=pltpu.MemorySpace.HBM),
            pl.BlockSpec(memory_space=pltpu.MemorySpace.VMEM),
            pl.BlockSpec(memory_space=pltpu.MemorySpace.VMEM),
        ],
        out_specs=pl.BlockSpec(memory_space=pltpu.MemorySpace.VMEM),
        out_shape=jax.ShapeDtypeStruct((B, N), jnp.float32),
        scratch_shapes=[
            pltpu.VMEM((_DEPTH, _ROWS, K), jnp.float32),
            pltpu.SemaphoreType.DMA((_DEPTH,)),
        ],
    )(inputs, mask, table)


# speedup vs baseline: 4.7848x; 4.7848x over previous
import jax
import jax.numpy as jnp
from jax.experimental import pallas as pl
from jax.experimental.pallas import tpu as pltpu

_BN = 2048


def _qtab_kernel(in_ref, mask_ref, table_ref, out_ref):
    a = table_ref[...].astype(jnp.bfloat16)
    b = in_ref[...].astype(jnp.bfloat16)
    out_ref[...] = (
        jnp.dot(a, b, preferred_element_type=jnp.float32) + mask_ref[...]
    )


def kernel(inputs, mask, table):
    B, K = inputs.shape
    N = table.shape[1]
    out_t = pl.pallas_call(
        _qtab_kernel,
        grid=(B // _BN,),
        in_specs=[
            pl.BlockSpec((K, _BN), lambda i: (0, i)),
            pl.BlockSpec((N, _BN), lambda i: (0, i)),
            pl.BlockSpec((N, K), lambda i: (0, 0)),
        ],
        out_specs=pl.BlockSpec((N, _BN), lambda i: (0, i)),
        out_shape=jax.ShapeDtypeStruct((N, B), jnp.float32),
        compiler_params=pltpu.CompilerParams(
            dimension_semantics=("arbitrary",),
        ),
    )(inputs.T, mask.T, table.T)
    return out_t.T
